# Initial kernel scaffold; baseline (speedup 1.0000x reference)
import jax
import jax.numpy as jnp
from jax.experimental import pallas as pl


def kernel(inputs, labels):
    raise NotImplementedError("write your pallas kernel here")


# speedup vs baseline: 5.0877x; 5.0877x over previous
import functools

import jax
import jax.numpy as jnp
from jax import lax
from jax.experimental import pallas as pl
from jax.experimental.pallas import tpu as pltpu
from jax.experimental.pallas import tpu_sc as plsc

_N = 100000
_C = 512
_NC = 2
_NS = 16
_NW = _NC * _NS
_L = 16
_CHUNK = 3136
_NPAD = _NW * _CHUNK
_VECS = _CHUNK // _L
_COLS = _C // _NS

_mesh = plsc.VectorSubcoreMesh(core_axis_name="c", subcore_axis_name="s",
                               num_cores=_NC, num_subcores=_NS)


@functools.partial(
    pl.kernel,
    out_type=jax.ShapeDtypeStruct((_NC, _C), jnp.float32),
    mesh=_mesh,
    scratch_types=[
        pltpu.VMEM((_CHUNK,), jnp.float32),
        pltpu.VMEM((_CHUNK,), jnp.int32),
        pltpu.VMEM((_C,), jnp.float32),
        pltpu.VMEM((_NS, _C), jnp.float32),
        pltpu.VMEM((_COLS,), jnp.float32),
        pltpu.VMEM_SHARED((_NS, _C), jnp.float32),
    ],
    compiler_params=pltpu.CompilerParams(needs_layout_passes=False),
)
def _seg_sum_sc(inputs_hbm, labels_hbm, out_hbm,
                vals_v, labs_v, acc_v, all_v, out_v, shared):
    cid = lax.axis_index("c")
    sid = lax.axis_index("s")
    wid = sid * _NC + cid
    base = wid * _CHUNK

    pltpu.sync_copy(inputs_hbm.at[pl.ds(base, _CHUNK)], vals_v)
    pltpu.sync_copy(labels_hbm.at[pl.ds(base, _CHUNK)], labs_v)

    zeros = jnp.zeros((_L,), jnp.float32)
    for j in range(_C // _L):
        acc_v[pl.ds(j * _L, _L)] = zeros

    def body(i, carry):
        lv = labs_v[pl.ds(i * _L, _L)]
        vv = vals_v[pl.ds(i * _L, _L)]
        plsc.addupdate_scatter(acc_v, [lv], vv)
        return carry

    lax.fori_loop(0, _VECS, body, 0, unroll=4)

    pltpu.sync_copy(acc_v, shared.at[sid])
    plsc.subcore_barrier()
    pltpu.sync_copy(shared, all_v)

    col0 = sid * _COLS

    def rbody(r, carry):
        a0, a1 = carry
        a0 = a0 + all_v[r, pl.ds(col0, _L)]
        a1 = a1 + all_v[r, pl.ds(col0 + _L, _L)]
        return (a0, a1)

    a0, a1 = lax.fori_loop(0, _NS, rbody, (zeros, zeros))
    out_v[pl.ds(0, _L)] = a0
    out_v[pl.ds(_L, _L)] = a1
    pltpu.sync_copy(out_v, out_hbm.at[cid, pl.ds(col0, _COLS)])


def kernel(inputs, labels):
    pad = _NPAD - _N
    inputs_p = jnp.concatenate([inputs, jnp.zeros((pad,), jnp.float32)])
    labels_p = jnp.concatenate(
        [labels.astype(jnp.int32), jnp.zeros((pad,), jnp.int32)])
    partial = _seg_sum_sc(inputs_p, labels_p)
    return partial[0] + partial[1]
